# baseline (speedup 1.0000x reference)
import functools

import jax
import jax.numpy as jnp
from jax import lax
from jax.experimental import pallas as pl
from jax.experimental.pallas import tpu as pltpu
from jax.experimental.pallas import tpu_sc as plsc

B = 16384
EMB = 32
NC = 2
NS = 16
NW = NC * NS
BPW = B // NW
CH = 128
NCHUNK = BPW // CH


def _gather_body(user_tab, store_tab, uid, sid, ue_out, se_out,
                 uidx_v, sidx_v, urows_v, srows_v, sem):
    wid = lax.axis_index("s") * NC + lax.axis_index("c")
    base = wid * BPW
    pltpu.sync_copy(uid.at[wid], uidx_v)
    pltpu.sync_copy(sid.at[wid], sidx_v)
    copies = []
    for j in range(NCHUNK):
        copies.append(pltpu.async_copy(
            user_tab.at[uidx_v.at[j]], urows_v.at[pl.ds(j * CH, CH)], sem))
        copies.append(pltpu.async_copy(
            store_tab.at[sidx_v.at[j]], srows_v.at[pl.ds(j * CH, CH)], sem))
    for c in copies:
        c.wait()
    pltpu.sync_copy(urows_v, ue_out.at[pl.ds(base, BPW)])
    pltpu.sync_copy(srows_v, se_out.at[pl.ds(base, BPW)])


_sc_gather = pl.kernel(
    _gather_body,
    out_type=(
        jax.ShapeDtypeStruct((B, EMB), jnp.float32),
        jax.ShapeDtypeStruct((B, EMB), jnp.float32),
    ),
    mesh=plsc.VectorSubcoreMesh(core_axis_name="c", subcore_axis_name="s"),
    scratch_types=[
        pltpu.VMEM((NCHUNK, CH), jnp.int32),
        pltpu.VMEM((NCHUNK, CH), jnp.int32),
        pltpu.VMEM((BPW, EMB), jnp.float32),
        pltpu.VMEM((BPW, EMB), jnp.float32),
        pltpu.SemaphoreType.DMA,
    ],
    compiler_params=pltpu.CompilerParams(use_tc_tiling_on_sc=False),
)


def _mlp_body(ue, se, f, w1u, w1s, w1f, b1, w2, b2, w3, b3, out):
    h = jnp.dot(ue[...], w1u[...], preferred_element_type=jnp.float32)
    h += jnp.dot(se[...], w1s[...], preferred_element_type=jnp.float32)
    h += jnp.dot(f[...], w1f[...], preferred_element_type=jnp.float32)
    h = jnp.maximum(h + b1[...], 0.0)
    h2 = jnp.dot(h, w2[...], preferred_element_type=jnp.float32)
    h2 = jnp.maximum(h2 + b2[...], 0.0)
    o = jnp.dot(h2, w3[...], preferred_element_type=jnp.float32) + b3[...]
    out[...] = 1.0 / (1.0 + jnp.exp(-o))


@functools.partial(jax.jit, static_argnames=())
def kernel(user_id, store_id, sentiment, rating, distance, hour_sin,
           user_table, store_table, W1, b1, W2, b2, W3, b3):
    uid = user_id.astype(jnp.int32).reshape(NW, NCHUNK, CH)
    sid = store_id.astype(jnp.int32).reshape(NW, NCHUNK, CH)
    ue, se = _sc_gather(user_table, store_table, uid, sid)

    f = jnp.stack([sentiment, rating, distance, hour_sin], axis=1)
    w1u = W1[:EMB]
    w1s = W1[EMB:2 * EMB]
    w1f = W1[2 * EMB:]

    out = pl.pallas_call(
        _mlp_body,
        out_shape=jax.ShapeDtypeStruct((B, 1), jnp.float32),
    )(ue, se, f, w1u, w1s, w1f,
      b1.reshape(1, 64), W2, b2.reshape(1, 32), W3, b3.reshape(1, 1))
    return out.reshape(B)
